# baseline (speedup 1.0000x reference)
import functools

import jax
import jax.numpy as jnp
from jax import lax
from jax.experimental import pallas as pl
from jax.experimental.pallas import tpu as pltpu
from jax.experimental.pallas import tpu_sc as plsc

B = 16384
D = 32
NC = 2
NS = 16
NW = NC * NS
BPW = B // NW
NCHUNK = 4
CH = BPW // NCHUNK
GROUPS = BPW // 16


def _fpmc_body(u_hbm, l_hbm, n_hbm, ui_hbm, iu_hbm, li_hbm, il_hbm, out_hbm,
               u_v, l_v, n_v, ui_v, iu_v, li_v, il_v, out_v, sem):
    wid = lax.axis_index("s") * NC + lax.axis_index("c")

    pltpu.sync_copy(u_hbm.at[wid], u_v)
    pltpu.sync_copy(l_hbm.at[wid], l_v)
    pltpu.sync_copy(n_hbm.at[wid], n_v)

    copies = []
    for j in range(NCHUNK):
        dst = pl.ds(j * CH, CH)
        copies.append(pltpu.async_copy(ui_hbm.at[u_v.at[j]], ui_v.at[dst], sem))
        copies.append(pltpu.async_copy(iu_hbm.at[n_v.at[j]], iu_v.at[dst], sem))
        copies.append(pltpu.async_copy(li_hbm.at[l_v.at[j]], li_v.at[dst], sem))
        copies.append(pltpu.async_copy(il_hbm.at[n_v.at[j]], il_v.at[dst], sem))
    for c in copies:
        c.wait()

    iota16 = lax.iota(jnp.int32, 16)

    def group(g, carry):
        row = g * 16 + iota16
        acc = jnp.zeros((16,), jnp.float32)
        for j in range(D):
            col = jnp.full((16,), j, jnp.int32)
            a = plsc.load_gather(ui_v, [row, col])
            b = plsc.load_gather(iu_v, [row, col])
            c = plsc.load_gather(il_v, [row, col])
            d = plsc.load_gather(li_v, [row, col])
            acc = acc + a * b + c * d
        sig = 1.0 / (1.0 + jnp.exp(-acc))
        out_v[pl.ds(pl.multiple_of(g * 16, 16), 16)] = sig
        return carry

    lax.fori_loop(0, GROUPS, group, 0)

    base = pl.multiple_of(wid * BPW, BPW)
    pltpu.sync_copy(out_v, out_hbm.at[pl.ds(base, BPW)])


_fpmc = functools.partial(
    pl.kernel,
    out_type=jax.ShapeDtypeStruct((B,), jnp.float32),
    mesh=plsc.VectorSubcoreMesh(core_axis_name="c", subcore_axis_name="s"),
    compiler_params=pltpu.CompilerParams(
        needs_layout_passes=False, use_tc_tiling_on_sc=False),
    scratch_types=[
        pltpu.VMEM((NCHUNK, CH), jnp.int32),
        pltpu.VMEM((NCHUNK, CH), jnp.int32),
        pltpu.VMEM((NCHUNK, CH), jnp.int32),
        pltpu.VMEM((BPW, D), jnp.float32),
        pltpu.VMEM((BPW, D), jnp.float32),
        pltpu.VMEM((BPW, D), jnp.float32),
        pltpu.VMEM((BPW, D), jnp.float32),
        pltpu.VMEM((BPW,), jnp.float32),
        pltpu.SemaphoreType.DMA,
    ],
)(_fpmc_body)


def kernel(user_id, item_last_click, next_item, UI, IU, LI, IL):
    u = user_id.reshape(NW, NCHUNK, CH).astype(jnp.int32)
    l = item_last_click.reshape(NW, NCHUNK, CH).astype(jnp.int32)
    n = next_item.reshape(NW, NCHUNK, CH).astype(jnp.int32)
    return _fpmc(u, l, n, UI, IU, LI, IL)
